# Initial kernel scaffold; baseline (speedup 1.0000x reference)
import jax
import jax.numpy as jnp
from jax.experimental import pallas as pl


def kernel(feats, points0, points1, points2, points3, hsv0, hsv1, hsv2, neighbors0, neighbors1, neighbors2, neighbors3, subsampling0, subsampling1, subsampling2, upsampling1, upsampling2, params):
    raise NotImplementedError("write your pallas kernel here")


# speedup vs baseline: 2.2334x; 2.2334x over previous
import functools

import numpy as np
import jax
import jax.numpy as jnp
from jax import lax
from jax.experimental import pallas as pl
from jax.experimental.pallas import tpu as pltpu
from jax.experimental.pallas import tpu_sc as plsc

GN = 32
KSZ = 15
SLOPE = 0.1
SIG = 0.05
F32 = jnp.float32
_PET = dict(preferred_element_type=jnp.float32,
            precision=lax.Precision.HIGHEST)


def _leaky(x):
    return jnp.where(x >= 0, x, SLOPE * x)


def _b16(x):
    return x.astype(jnp.bfloat16).astype(F32)


def _dotb(a, b):
    return jnp.dot(_b16(a), _b16(b), **_PET)


def _sds(shape):
    return jax.ShapeDtypeStruct(shape, F32)


def _sc_gather(table, idx):
    M, D = table.shape
    B = idx.shape[0]
    NW = 32
    CH = 128
    nchunks = B // CH
    nloop = -(-nchunks // NW)
    mesh = plsc.VectorSubcoreMesh(core_axis_name="c", subcore_axis_name="s")

    @functools.partial(
        pl.kernel,
        mesh=mesh,
        out_type=_sds((B, D)),
        scratch_types=[
            pltpu.VMEM((CH,), jnp.int32),
            pltpu.VMEM((CH, D), F32),
            pltpu.SemaphoreType.DMA,
        ],
    )
    def gk(tab_hbm, idx_hbm, out_hbm, idx_v, rows_v, sem):
        wid = lax.axis_index("s") * 2 + lax.axis_index("c")

        def body(i, carry):
            cid = i * NW + wid

            @pl.when(cid < nchunks)
            def _():
                off = cid * CH
                pltpu.sync_copy(idx_hbm.at[pl.ds(off, CH)], idx_v)
                pltpu.async_copy(tab_hbm.at[idx_v], rows_v, sem).wait()
                pltpu.sync_copy(rows_v, out_hbm.at[pl.ds(off, CH)])

            return carry

        lax.fori_loop(0, nloop, body, 0)

    return gk(table, idx)


_GM_CACHE = {}


def _gmats(c):
    if c not in _GM_CACHE:
        cg = c // GN
        gm = np.zeros((c, GN), np.float32)
        gm[np.arange(c), np.arange(c) // cg] = 1.0
        _GM_CACHE[c] = (jnp.asarray(gm), jnp.asarray(gm.T))
    return _GM_CACHE[c]


def _gn_in(y, gm, gmt, g, bt, n, cg):
    cnt = float(n * cg)
    s = jnp.dot(jnp.sum(y, axis=0, keepdims=True), gm, **_PET)
    ss = jnp.dot(jnp.sum(y * y, axis=0, keepdims=True), gm, **_PET)
    m = s / cnt
    v = ss / cnt - m * m
    mf = jnp.dot(m, gmt, **_PET)
    vf = jnp.dot(v, gmt, **_PET)
    return (y - mf) * lax.rsqrt(vf + 1e-5) * g + bt


def _unary_full(x, w, b, g=None, bt=None, relu=True):
    n = x.shape[0]
    c = w.shape[1]
    if g is None:
        def body(x_ref, w_ref, b_ref, o_ref):
            y = _dotb(x_ref[...], w_ref[...]) + b_ref[...]
            o_ref[...] = _leaky(y) if relu else y

        return pl.pallas_call(body, out_shape=_sds((n, c)))(
            x, w, b.reshape(1, c))

    gm, gmt = _gmats(c)

    def body(x_ref, w_ref, b_ref, gm_ref, gmt_ref, g_ref, bt_ref, o_ref):
        y = _dotb(x_ref[...], w_ref[...]) + b_ref[...]
        y = _gn_in(y, gm_ref[...], gmt_ref[...], g_ref[...], bt_ref[...],
                   n, c // GN)
        o_ref[...] = _leaky(y) if relu else y

    return pl.pallas_call(body, out_shape=_sds((n, c)))(
        x, w, b.reshape(1, c), gm, gmt, g.reshape(1, c), bt.reshape(1, c))


def _gn_act_full(y, g, bt):
    n, c = y.shape
    gm, gmt = _gmats(c)

    def body(y_ref, gm_ref, gmt_ref, g_ref, bt_ref, o_ref):
        z = _gn_in(y_ref[...], gm_ref[...], gmt_ref[...], g_ref[...],
                   bt_ref[...], n, c // GN)
        o_ref[...] = _leaky(z)

    return pl.pallas_call(body, out_shape=_sds((n, c)))(
        y, gm, gmt, g.reshape(1, c), bt.reshape(1, c))


def _res_tail(xmid, w2, b2, g2, bt2, sc_in, wsc=None, bsc=None, gsc=None,
              btsc=None):
    n = xmid.shape[0]
    c = w2.shape[1]
    gm, gmt = _gmats(c)
    cg = c // GN

    if wsc is None:
        def body(x_ref, w_ref, b_ref, gm_ref, gmt_ref, g_ref, bt_ref,
                 s_ref, o_ref):
            y = _dotb(x_ref[...], w_ref[...]) + b_ref[...]
            y = _gn_in(y, gm_ref[...], gmt_ref[...], g_ref[...], bt_ref[...],
                       n, cg)
            o_ref[...] = _leaky(y + s_ref[...])

        return pl.pallas_call(body, out_shape=_sds((n, c)))(
            xmid, w2, b2.reshape(1, c), gm, gmt, g2.reshape(1, c),
            bt2.reshape(1, c), sc_in)

    def body(x_ref, w_ref, b_ref, gm_ref, gmt_ref, g_ref, bt_ref, s_ref,
             ws_ref, bs_ref, gs_ref, bts_ref, o_ref):
        y = _dotb(x_ref[...], w_ref[...]) + b_ref[...]
        y = _gn_in(y, gm_ref[...], gmt_ref[...], g_ref[...], bt_ref[...],
                   n, cg)
        ys = _dotb(s_ref[...], ws_ref[...]) + bs_ref[...]
        ys = _gn_in(ys, gm_ref[...], gmt_ref[...], gs_ref[...], bts_ref[...],
                    n, cg)
        o_ref[...] = _leaky(y + ys)

    return pl.pallas_call(body, out_shape=_sds((n, c)))(
        xmid, w2, b2.reshape(1, c), gm, gmt, g2.reshape(1, c),
        bt2.reshape(1, c), sc_in, wsc, bsc.reshape(1, c), gsc.reshape(1, c),
        btsc.reshape(1, c))


def _tile(n, f):
    t = 1
    budget = max(1, (6 * 1024 * 1024) // (32 * f * 4))
    while t * 2 <= n and t * 2 <= budget:
        t *= 2
    return t


def _conv_call(ngath, qp, kp, wk, sigma):
    nk, fdim = ngath.shape
    n = qp.shape[0]
    k = nk // n
    c = wk.shape[1]
    d = wk.shape[2]
    g3 = ngath.reshape(n, k, fdim)
    qp3 = qp.reshape(n, 1, 16)
    kpp = jnp.pad(kp, ((0, 1), (0, 0)))
    kpx = kpp[:, 0].reshape(1, 1, 16)
    kpy = kpp[:, 1].reshape(1, 1, 16)
    kpz = kpp[:, 2].reshape(1, 1, 16)
    kp2 = jnp.sum(kpp * kpp, axis=1).reshape(1, 1, 16)
    wkflat = wk.reshape(KSZ * c, d)
    tn = _tile(n, fdim)
    inv_sig = 1.0 / sigma

    def body(g_ref, q_ref, kx_ref, ky_ref, kz_ref, k2_ref, w_ref, o_ref):
        g = g_ref[...]
        rel = g[:, :, :16] - q_ref[...]
        rel2 = jnp.sum(rel * rel, axis=2, keepdims=True)
        dots = (rel[:, :, 0:1] * kx_ref[...] + rel[:, :, 1:2] * ky_ref[...]
                + rel[:, :, 2:3] * kz_ref[...])
        dist = jnp.sqrt(jnp.maximum(rel2 - 2.0 * dots + k2_ref[...], 0.0))
        infl = jnp.maximum(0.0, 1.0 - dist * inv_sig)
        nf = _b16(g[:, :, 16:16 + c])
        inflr = _b16(infl)
        parts = [jnp.sum(inflr[:, :, p:p + 1] * nf, axis=1)
                 for p in range(KSZ)]
        agg = jnp.concatenate(parts, axis=1)
        o_ref[...] = _dotb(agg, w_ref[...])

    small = pl.BlockSpec((1, 1, 16), lambda i: (0, 0, 0))
    return pl.pallas_call(
        body,
        grid=(n // tn,),
        in_specs=[
            pl.BlockSpec((tn, k, fdim), lambda i: (i, 0, 0)),
            pl.BlockSpec((tn, 1, 16), lambda i: (i, 0, 0)),
            small, small, small, small,
            pl.BlockSpec((KSZ * c, d), lambda i: (0, 0)),
        ],
        out_specs=pl.BlockSpec((tn, d), lambda i: (i, 0)),
        out_shape=_sds((n, d)),
    )(g3, qp3, kpx, kpy, kpz, kp2, wkflat)


def _maxpool_call(gf, n, k):
    c = gf.shape[1]
    g3 = gf.reshape(n, k, c)
    tn = _tile(n, c)

    def body(g_ref, o_ref):
        o_ref[...] = jnp.max(g_ref[...], axis=1)

    return pl.pallas_call(
        body,
        grid=(n // tn,),
        in_specs=[pl.BlockSpec((tn, k, c), lambda i: (i, 0, 0))],
        out_specs=pl.BlockSpec((tn, c), lambda i: (i, 0)),
        out_shape=_sds((n, c)),
    )(g3)


def _bm_call(hg, w1, b1, w2, b2):
    n, cin = hg.shape
    tn = 4096

    def body(h_ref, w1_ref, b1_ref, w2_ref, b2_ref, o_ref):
        h = h_ref[...]
        t = jax.nn.gelu(_dotb(h, w1_ref[...]) + b1_ref[...])
        o_ref[...] = h[:, :128] + _dotb(t, w2_ref[...]) + b2_ref[...]

    return pl.pallas_call(
        body,
        grid=(n // tn,),
        in_specs=[
            pl.BlockSpec((tn, cin), lambda i: (i, 0)),
            pl.BlockSpec(w1.shape, lambda i: (0, 0)),
            pl.BlockSpec((1, 256), lambda i: (0, 0)),
            pl.BlockSpec(w2.shape, lambda i: (0, 0)),
            pl.BlockSpec((1, 128), lambda i: (0, 0)),
        ],
        out_specs=pl.BlockSpec((tn, 128), lambda i: (i, 0)),
        out_shape=_sds((n, 128)),
    )(hg, w1, b1.reshape(1, 256), w2, b2.reshape(1, 128))


def _pad16(x):
    c = x.shape[1]
    pad = (-c) % 16
    return jnp.pad(x, ((0, 0), (0, pad))) if pad else x


def _conv_block(p, s_feats, qp_pad, sp_pad, nidx_flat, nq, sigma):
    table = jnp.concatenate([sp_pad, _pad16(s_feats)], axis=1)
    fpad = (-table.shape[1]) % 128
    if fpad:
        table = jnp.pad(table, ((0, 0), (0, fpad)))
    gath = _sc_gather(table, nidx_flat)
    cin = s_feats.shape[1]
    wk = p["wk"]
    pad = (-cin) % 16
    if pad:
        wk = jnp.pad(wk, ((0, 0), (0, pad), (0, 0)))
    return _conv_call(gath, qp_pad, p["kp"], wk, sigma)


def _res_block(p, s_feats, qp_pad, sp_pad, nidx_flat, nq, sigma,
               strided=False):
    x = _unary_full(s_feats, p["u1"]["w"], p["u1"]["b"], p["u1"]["g"],
                    p["u1"]["bt"], relu=True)
    y = _conv_block(p["conv"], x, qp_pad, sp_pad, nidx_flat, nq, sigma)
    y = _gn_act_full(y, p["conv"]["g"], p["conv"]["bt"])
    if strided:
        sc = _maxpool_call(_sc_gather(s_feats, nidx_flat), nq, 32)
    else:
        sc = s_feats
    if "sc" in p:
        return _res_tail(y, p["u2"]["w"], p["u2"]["b"], p["u2"]["g"],
                         p["u2"]["bt"], sc, p["sc"]["w"], p["sc"]["b"],
                         p["sc"]["g"], p["sc"]["bt"])
    return _res_tail(y, p["u2"]["w"], p["u2"]["b"], p["u2"]["g"],
                     p["u2"]["bt"], sc)


def _enhance(f, color):
    return jnp.concatenate(
        [f, color[:, 0:1], color[:, 2:3], color[:, 1:2]], axis=1)


def kernel(feats, points0, points1, points2, points3, hsv0, hsv1, hsv2,
           neighbors0, neighbors1, neighbors2, neighbors3, subsampling0,
           subsampling1, subsampling2, upsampling1, upsampling2, params):
    n0, n1, n2, n3 = points0.shape[0], points1.shape[0], points2.shape[0], points3.shape[0]
    p0 = _pad16(points0)
    p1 = _pad16(points1)
    p2 = _pad16(points2)
    p3 = _pad16(points3)
    nb0 = neighbors0.reshape(-1).astype(jnp.int32)
    nb1 = neighbors1.reshape(-1).astype(jnp.int32)
    nb2 = neighbors2.reshape(-1).astype(jnp.int32)
    nb3 = neighbors3.reshape(-1).astype(jnp.int32)
    sb0 = subsampling0.reshape(-1).astype(jnp.int32)
    sb1 = subsampling1.reshape(-1).astype(jnp.int32)
    sb2 = subsampling2.reshape(-1).astype(jnp.int32)

    f1 = _enhance(feats, hsv0)
    y = _conv_block(params["enc1_1"], f1, p0, p0, nb0, n0, SIG)
    f1 = _gn_act_full(y, params["enc1_1"]["g"], params["enc1_1"]["bt"])
    f1 = _res_block(params["enc1_2"], f1, p0, p0, nb0, n0, SIG)

    bm = params["bm"]
    hg = jnp.concatenate([f1, points0, hsv0[:, 0:1], hsv0[:, 2:3],
                          hsv0[:, 1:2]], axis=1)
    hg = jnp.pad(hg, ((0, 0), (0, 10)))
    w1p = jnp.pad(bm["w1"], ((0, 10), (0, 0)))
    f1 = _bm_call(hg, w1p, bm["b1"], bm["w2"], bm["b2"])

    f2 = _res_block(params["enc2_1"], f1, p1, p0, sb0, n1, SIG, strided=True)
    f2 = _enhance(f2, hsv1)
    f2 = _res_block(params["enc2_2"], f2, p1, p1, nb1, n1, SIG * 2)
    f2 = _res_block(params["enc2_3"], f2, p1, p1, nb1, n1, SIG * 2)

    f3 = _res_block(params["enc3_1"], f2, p2, p1, sb1, n2, SIG * 2,
                    strided=True)
    f3 = _enhance(f3, hsv2)
    f3 = _res_block(params["enc3_2"], f3, p2, p2, nb2, n2, SIG * 4)
    f3 = _res_block(params["enc3_3"], f3, p2, p2, nb2, n2, SIG * 4)

    f4 = _res_block(params["enc4_1"], f3, p3, p2, sb2, n3, SIG * 4,
                    strided=True)
    f4 = _res_block(params["enc4_2"], f4, p3, p3, nb3, n3, SIG * 8)
    f4 = _res_block(params["enc4_3"], f4, p3, p3, nb3, n3, SIG * 8)

    u2 = upsampling2[:, 0].astype(jnp.int32)
    g4 = jnp.concatenate(
        [_sc_gather(f4[:, :512], u2), _sc_gather(f4[:, 512:], u2)], axis=1)
    l3 = _unary_full(jnp.concatenate([g4, f3], axis=1), params["dec3"]["w"],
                     params["dec3"]["b"], params["dec3"]["g"],
                     params["dec3"]["bt"], relu=True)
    g3 = _sc_gather(l3, upsampling1[:, 0].astype(jnp.int32))
    l2 = _unary_full(jnp.concatenate([g3, f2], axis=1), params["dec2"]["w"],
                     params["dec2"]["b"], relu=False)
    return (l2, l3, f4)
